# baseline (speedup 1.0000x reference)
import functools

import jax
import jax.numpy as jnp
from jax import lax
from jax.experimental import pallas as pl
from jax.experimental.pallas import tpu as pltpu
from jax.experimental.pallas import tpu_sc as plsc

N = 10000
E = 320000
D = 128

NC = 2
NS = 16
NW = NC * NS
K = 128
NBUF = 2
SLAB = 20
NSLAB_TOT = 128
H_SLABS = 4
L_SLABS = 4
CF = 0
E_PAD = NSLAB_TOT * SLAB * K
ZK = 80
ZCH = N // ZK


def _sc_aggregate(x, rows4, cols4, vals4):
    mesh = plsc.VectorSubcoreMesh(core_axis_name="c", subcore_axis_name="s")

    @functools.partial(
        pl.kernel,
        mesh=mesh,
        out_type=jax.ShapeDtypeStruct((NC, N, D), jnp.float32),
        scratch_types=[
            pltpu.VMEM((SLAB, K), jnp.int32),
            pltpu.VMEM((SLAB, K), jnp.int32),
            pltpu.VMEM((SLAB, K), jnp.float32),
            [pltpu.VMEM((K, D), jnp.float32) for _ in range(NBUF)],
            pltpu.VMEM_SHARED((N, D), jnp.float32),
            [pltpu.SemaphoreType.DMA for _ in range(NBUF)],
        ],
    )
    def k(x_hbm, rows_hbm, cols_hbm, vals_hbm, out_hbm,
          rows_v, cols_v, vals_v, gbufs, acc, sems):
        c = lax.axis_index("c")
        s = lax.axis_index("s")

        zero = jnp.zeros((16,), jnp.float32)
        gb0 = gbufs[0]

        def zbody(i, _):
            gb0[i // (D // 16), pl.ds((i % (D // 16)) * 16, 16)] = zero
            return 0

        lax.fori_loop(0, ZK * (D // 16), zbody, 0)
        for t in range((ZCH + NS - 1) // NS):
            i = s + t * NS

            @pl.when(i < ZCH)
            def _():
                pltpu.sync_copy(gb0.at[pl.ds(0, ZK)], acc.at[pl.ds(i * ZK, ZK)])

        plsc.subcore_barrier()

        dn = lax.GatherDimensionNumbers(
            offset_dims=(), collapsed_slice_dims=(0,), start_index_map=(0,))

        def scale(gbuf, jj):
            def gbody(g, _):
                vv = vals_v[jj, pl.ds(g * 16, 16)]
                for e in range(16):
                    r = g * 16 + e
                    sp = lax.gather(
                        vv, jnp.full((16, 1), e, jnp.int32), dn, (1,),
                        mode=lax.GatherScatterMode.PROMISE_IN_BOUNDS)
                    for f in range(D // 16):
                        gbuf[r, pl.ds(f * 16, 16)] = (
                            gbuf[r, pl.ds(f * 16, 16)] * sp)
                return 0

            lax.fori_loop(0, K // 16, gbody, 0)

        base = lax.select(c == CF, NS * L_SLABS + s * H_SLABS, s * L_SLABS)
        nsl = lax.select(c == CF, H_SLABS, L_SLABS)

        def slab_loop(t, _):
            sid = base + t
            pltpu.sync_copy(rows_hbm.at[sid], rows_v)
            pltpu.sync_copy(cols_hbm.at[sid], cols_v)
            pltpu.sync_copy(vals_hbm.at[sid], vals_v)

            for u in range(NBUF):
                pltpu.async_copy(x_hbm.at[cols_v.at[u]], gbufs[u], sems[u])

            def batch(b4, _):
                jj = b4 * NBUF
                for u in range(NBUF):
                    pltpu.make_async_copy(
                        x_hbm.at[cols_v.at[jj + u]], gbufs[u], sems[u]).wait()
                    scale(gbufs[u], jj + u)
                    pltpu.sync_copy(gbufs[u], acc.at[rows_v.at[jj + u]],
                                    add=True)

                    @pl.when(jj + u + NBUF < SLAB)
                    def _():
                        pltpu.async_copy(
                            x_hbm.at[cols_v.at[jj + u + NBUF]],
                            gbufs[u], sems[u])
                return 0

            lax.fori_loop(0, SLAB // NBUF, batch, 0)
            return 0

        lax.fori_loop(0, nsl, slab_loop, 0)

        plsc.subcore_barrier()

        for t in range((ZCH + NS - 1) // NS):
            i = s + t * NS

            @pl.when(i < ZCH)
            def _():
                pltpu.sync_copy(acc.at[pl.ds(i * ZK, ZK)],
                                out_hbm.at[c, pl.ds(i * ZK, ZK)])

    return k(x, rows4, cols4, vals4)


def _tc_finish(parts, W0):
    BM = 1000

    def body(p_ref, w_ref, o_ref):
        ps = p_ref[0] + p_ref[1]
        o_ref[...] = jnp.maximum(
            jnp.dot(ps, w_ref[...], preferred_element_type=jnp.float32), 0.0)

    return pl.pallas_call(
        body,
        grid=(N // BM,),
        in_specs=[
            pl.BlockSpec((NC, BM, D), lambda i: (0, i, 0)),
            pl.BlockSpec((D, D), lambda i: (0, 0)),
        ],
        out_specs=pl.BlockSpec((BM, D), lambda i: (i, 0)),
        out_shape=jax.ShapeDtypeStruct((N, D), jnp.float32),
    )(parts, W0)


@jax.jit
def kernel(x, edge_index, edge_vals, W0):
    order = jnp.argsort(edge_index[1])
    rows = edge_index[0][order]
    cols = edge_index[1][order]
    vals = edge_vals[order]
    pad = E_PAD - E
    rows4 = jnp.concatenate(
        [rows, jnp.zeros((pad,), edge_index.dtype)]
    ).reshape(NSLAB_TOT, SLAB, K)
    cols4 = jnp.concatenate(
        [cols, jnp.zeros((pad,), edge_index.dtype)]
    ).reshape(NSLAB_TOT, SLAB, K)
    vals4 = jnp.concatenate(
        [vals, jnp.zeros((pad,), edge_vals.dtype)]
    ).reshape(NSLAB_TOT, SLAB, K)
    parts = _sc_aggregate(x, rows4, cols4, vals4)
    return _tc_finish(parts, W0)


# speedup vs baseline: 1.7170x; 1.7170x over previous
import functools

import jax
import jax.numpy as jnp
from jax import lax
from jax.experimental import pallas as pl
from jax.experimental.pallas import tpu as pltpu
from jax.experimental.pallas import tpu_sc as plsc

N = 10000
E = 320000
D = 128

NC = 2
NS = 16
NW = NC * NS
K = 128
NBUF = 2
SLAB = 20
NSLAB_TOT = 128
H_SLABS = 4
L_SLABS = 4
CF = 0
E_PAD = NSLAB_TOT * SLAB * K
ZK = 80
ZCH = N // ZK


def _sc_aggregate(x, rows4, cols4, vals4):
    mesh = plsc.VectorSubcoreMesh(core_axis_name="c", subcore_axis_name="s")

    @functools.partial(
        pl.kernel,
        mesh=mesh,
        out_type=jax.ShapeDtypeStruct((NC, N, D), jnp.float32),
        scratch_types=[
            pltpu.VMEM((SLAB, K), jnp.int32),
            pltpu.VMEM((SLAB, K), jnp.int32),
            pltpu.VMEM((SLAB, K), jnp.float32),
            [pltpu.VMEM((K, D), jnp.float32) for _ in range(NBUF)],
            pltpu.VMEM_SHARED((N, D), jnp.float32),
            [pltpu.SemaphoreType.DMA for _ in range(NBUF)],
        ],
    )
    def k(x_hbm, rows_hbm, cols_hbm, vals_hbm, out_hbm,
          rows_v, cols_v, vals_v, gbufs, acc, sems):
        c = lax.axis_index("c")
        s = lax.axis_index("s")

        zero = jnp.zeros((16,), jnp.float32)
        gb0 = gbufs[0]

        def zbody(i, _):
            gb0[i // (D // 16), pl.ds((i % (D // 16)) * 16, 16)] = zero
            return 0

        lax.fori_loop(0, ZK * (D // 16), zbody, 0)
        for t in range((ZCH + NS - 1) // NS):
            i = s + t * NS

            @pl.when(i < ZCH)
            def _():
                pltpu.sync_copy(gb0.at[pl.ds(0, ZK)], acc.at[pl.ds(i * ZK, ZK)])

        plsc.subcore_barrier()

        dn = lax.GatherDimensionNumbers(
            offset_dims=(), collapsed_slice_dims=(0,), start_index_map=(0,))

        def scale(gbuf, jj):
            def gbody(g, _):
                vv = vals_v[jj, pl.ds(g * 16, 16)]
                for e in range(16):
                    r = g * 16 + e
                    sp = lax.gather(
                        vv, jnp.full((16, 1), e, jnp.int32), dn, (1,),
                        mode=lax.GatherScatterMode.PROMISE_IN_BOUNDS)
                    for f in range(D // 16):
                        gbuf[r, pl.ds(f * 16, 16)] = (
                            gbuf[r, pl.ds(f * 16, 16)] * sp)
                return 0

            lax.fori_loop(0, K // 16, gbody, 0)

        base = lax.select(c == CF, NS * L_SLABS + s * H_SLABS, s * L_SLABS)
        nsl = lax.select(c == CF, H_SLABS, L_SLABS)

        def slab_loop(t, _):
            sid = base + t
            pltpu.sync_copy(rows_hbm.at[sid], rows_v)
            pltpu.sync_copy(cols_hbm.at[sid], cols_v)
            pltpu.sync_copy(vals_hbm.at[sid], vals_v)

            for u in range(NBUF):
                pltpu.async_copy(x_hbm.at[cols_v.at[u]], gbufs[u], sems[u])

            def batch(b4, _):
                jj = b4 * NBUF
                for u in range(NBUF):
                    pltpu.make_async_copy(
                        x_hbm.at[cols_v.at[jj + u]], gbufs[u], sems[u]).wait()
                    scale(gbufs[u], jj + u)
                    pltpu.sync_copy(gbufs[u], acc.at[rows_v.at[jj + u]],
                                    add=True)

                    @pl.when(jj + u + NBUF < SLAB)
                    def _():
                        pltpu.async_copy(
                            x_hbm.at[cols_v.at[jj + u + NBUF]],
                            gbufs[u], sems[u])
                return 0

            lax.fori_loop(0, SLAB // NBUF, batch, 0)
            return 0

        lax.fori_loop(0, nsl, slab_loop, 0)

        plsc.subcore_barrier()

        for t in range((ZCH + NS - 1) // NS):
            i = s + t * NS

            @pl.when(i < ZCH)
            def _():
                pltpu.sync_copy(acc.at[pl.ds(i * ZK, ZK)],
                                out_hbm.at[c, pl.ds(i * ZK, ZK)])

    return k(x, rows4, cols4, vals4)


def _tc_finish(parts, W0):
    BM = 1000

    def body(p_ref, w_ref, o_ref):
        ps = p_ref[0] + p_ref[1]
        o_ref[...] = jnp.maximum(
            jnp.dot(ps, w_ref[...], preferred_element_type=jnp.float32), 0.0)

    return pl.pallas_call(
        body,
        grid=(N // BM,),
        in_specs=[
            pl.BlockSpec((NC, BM, D), lambda i: (0, i, 0)),
            pl.BlockSpec((D, D), lambda i: (0, 0)),
        ],
        out_specs=pl.BlockSpec((BM, D), lambda i: (i, 0)),
        out_shape=jax.ShapeDtypeStruct((N, D), jnp.float32),
    )(parts, W0)


@jax.jit
def kernel(x, edge_index, edge_vals, W0):
    pad = E_PAD - E
    rows4 = jnp.concatenate(
        [edge_index[0], jnp.zeros((pad,), edge_index.dtype)]
    ).reshape(NSLAB_TOT, SLAB, K)
    cols4 = jnp.concatenate(
        [edge_index[1], jnp.zeros((pad,), edge_index.dtype)]
    ).reshape(NSLAB_TOT, SLAB, K)
    vals4 = jnp.concatenate(
        [edge_vals, jnp.zeros((pad,), edge_vals.dtype)]
    ).reshape(NSLAB_TOT, SLAB, K)
    parts = _sc_aggregate(x, rows4, cols4, vals4)
    return _tc_finish(parts, W0)
